# baseline (speedup 1.0000x reference)
import functools

import jax
import jax.numpy as jnp
from jax import lax
from jax.experimental import pallas as pl
from jax.experimental.pallas import tpu as pltpu
from jax.experimental.pallas import tpu_sc as plsc

N = 10000
E = 320000
F_IN = 128
H = 64
C = 40
G = 64

NC = 2
NS = 16
NW = NC * NS

N_PAD = 10240
NR = N_PAD // NS
EC = 128
NCHUNK = 79
EWP = NCHUNK * EC
E_PAD = NW * EWP

R = 512
GRID = N_PAD // R

_mesh = plsc.VectorSubcoreMesh(core_axis_name="c", subcore_axis_name="s")
_sc_params = pltpu.CompilerParams(use_tc_tiling_on_sc=False)


DW = 16


@functools.partial(
    pl.kernel,
    mesh=_mesh,
    out_type=jax.ShapeDtypeStruct((NC, N_PAD, DW), jnp.float32),
    scratch_types=[
        pltpu.VMEM((NCHUNK, EC), jnp.int32),
        pltpu.VMEM((EC, DW), jnp.float32),
        pltpu.VMEM((NR, DW), jnp.float32),
        pltpu.VMEM_SHARED((N_PAD, DW), jnp.float32),
    ],
    compiler_params=_sc_params,
)
def _deg(dst_hbm, out_hbm, dstv, onesb, zbuf, acc):
    cid = lax.axis_index("c")
    sid = lax.axis_index("s")
    wid = sid * NC + cid
    pltpu.sync_copy(dst_hbm.at[wid], dstv)
    zv = jnp.zeros((16,), jnp.float32)
    ones = jnp.ones((16,), jnp.float32)

    def fill(r, carry):
        onesb[r, pl.ds(0, 16)] = ones
        return carry

    lax.fori_loop(0, EC, fill, 0)

    def zfill(r, carry):
        zbuf[r, pl.ds(0, 16)] = zv
        return carry

    lax.fori_loop(0, NR, zfill, 0)
    base = pl.multiple_of(sid * NR, 128)
    pltpu.sync_copy(zbuf, acc.at[pl.ds(base, NR)])
    plsc.subcore_barrier()

    def chunk(j, carry):
        pltpu.sync_copy(onesb, acc.at[dstv.at[j]], add=True)
        return carry

    lax.fori_loop(0, NCHUNK, chunk, 0)
    plsc.subcore_barrier()
    pltpu.sync_copy(acc.at[pl.ds(base, NR)], zbuf)
    pltpu.sync_copy(zbuf, out_hbm.at[cid, pl.ds(base, NR)])


@functools.partial(
    pl.kernel,
    mesh=_mesh,
    out_type=jax.ShapeDtypeStruct((NC, N_PAD, H), jnp.bfloat16),
    scratch_types=[
        pltpu.VMEM((NCHUNK, EC), jnp.int32),
        pltpu.VMEM((NCHUNK, EC), jnp.int32),
        pltpu.VMEM((2, EC, H), jnp.bfloat16),
        pltpu.VMEM((EC, H), jnp.bfloat16),
        pltpu.VMEM_SHARED((N_PAD, H), jnp.bfloat16),
        pltpu.VMEM_SHARED((N_PAD, H), jnp.bfloat16),
        pltpu.SemaphoreType.DMA,
    ],
    compiler_params=_sc_params,
)
def _agg(y_hbm, src_hbm, dst_hbm, out_hbm, srcv, dstv, rows, zbuf, acc, ysh,
         gsem):
    cid = lax.axis_index("c")
    sid = lax.axis_index("s")
    wid = sid * NC + cid
    pltpu.sync_copy(src_hbm.at[wid], srcv)
    pltpu.sync_copy(dst_hbm.at[wid], dstv)

    zv = jnp.zeros((32,), jnp.bfloat16)

    def zrow(r, carry):
        zbuf[r, pl.ds(0, 32)] = zv
        zbuf[r, pl.ds(32, 32)] = zv
        return carry

    lax.fori_loop(0, EC, zrow, 0)
    base = pl.multiple_of(sid * NR, 128)
    for k in range(NR // EC):
        pltpu.sync_copy(zbuf, acc.at[pl.ds(base + k * EC, EC)])
    pltpu.sync_copy(y_hbm.at[pl.ds(base, NR)], ysh.at[pl.ds(base, NR)])
    plsc.subcore_barrier()

    def chunk(j, carry):
        pltpu.async_copy(ysh.at[srcv.at[j]], rows.at[0], gsem).wait()
        pltpu.sync_copy(rows.at[0], acc.at[dstv.at[j]], add=True)
        return carry

    lax.fori_loop(0, NCHUNK, chunk, 0)
    plsc.subcore_barrier()

    pltpu.sync_copy(acc.at[pl.ds(base, NR)], out_hbm.at[cid, pl.ds(base, NR)])


def _dinv_block(deg_ref):
    a = deg_ref[...]
    degs = a[0, 0, :, :1] + a[1, 0, :, :1]
    return lax.rsqrt(1.0 + degs)


def _mmxw_body(x_ref, w_ref, xw_ref):
    xw_ref[...] = jnp.dot(x_ref[...], w_ref[...],
                          preferred_element_type=jnp.float32)


def _mmxw(xpad, W1):
    return pl.pallas_call(
        _mmxw_body,
        grid=(GRID,),
        in_specs=[
            pl.BlockSpec((R, F_IN), lambda i: (i, 0)),
            pl.BlockSpec((F_IN, H), lambda i: (0, 0)),
        ],
        out_specs=pl.BlockSpec((R, H), lambda i: (i, 0)),
        out_shape=jax.ShapeDtypeStruct((N_PAD, H), jnp.float32),
    )(xpad, W1)


def _scale_body(deg_ref, xw_ref, y_ref):
    dinv = _dinv_block(deg_ref)
    y_ref[...] = (xw_ref[...] * dinv).astype(jnp.bfloat16)


def _scale(xw, deg_parts):
    return pl.pallas_call(
        _scale_body,
        grid=(GRID,),
        in_specs=[
            pl.BlockSpec((NC, 1, R, DW), lambda i: (0, i, 0, 0)),
            pl.BlockSpec((R, H), lambda i: (i, 0)),
        ],
        out_specs=pl.BlockSpec((R, H), lambda i: (i, 0)),
        out_shape=jax.ShapeDtypeStruct((N_PAD, H), jnp.bfloat16),
    )(deg_parts, xw)


def _mid_body(deg_ref, agg_ref, y_ref, b_ref, w_ref, y2_ref):
    dinv = _dinv_block(deg_ref)
    a = agg_ref[...].astype(jnp.float32)
    y = y_ref[...].astype(jnp.float32)
    h = (a[0] + a[1] + y) * dinv + b_ref[...]
    h = jnp.maximum(h, 0.0)
    y2_ref[...] = (jnp.dot(h, w_ref[...],
                           preferred_element_type=jnp.float32)
                   * dinv).astype(jnp.bfloat16)


def _mid(deg_parts, agg, y1, b1, W2):
    return pl.pallas_call(
        _mid_body,
        grid=(GRID,),
        in_specs=[
            pl.BlockSpec((NC, 1, R, DW), lambda i: (0, i, 0, 0)),
            pl.BlockSpec((NC, R, H), lambda i: (0, i, 0)),
            pl.BlockSpec((R, H), lambda i: (i, 0)),
            pl.BlockSpec((1, H), lambda i: (0, 0)),
            pl.BlockSpec((H, H), lambda i: (0, 0)),
        ],
        out_specs=pl.BlockSpec((R, H), lambda i: (i, 0)),
        out_shape=jax.ShapeDtypeStruct((N_PAD, H), jnp.bfloat16),
    )(deg_parts, agg, y1, b1.reshape(1, H), W2)


def _fin_body(deg_ref, agg_ref, y_ref, b_ref, batch_ref, wl_ref, bl_ref,
              out_ref, s_acc, cnt_acc):
    i = pl.program_id(0)
    dinv = _dinv_block(deg_ref)
    a = agg_ref[...].astype(jnp.float32)
    y = y_ref[...].astype(jnp.float32)
    h = (a[0] + a[1] + y) * dinv + b_ref[...]
    h = jnp.maximum(h, 0.0)
    seg = lax.broadcasted_iota(jnp.int32, (1, G), 1)
    P = (batch_ref[...] == seg).astype(jnp.float32)
    s_part = lax.dot_general(P, h, (((0,), (0,)), ((), ())),
                             preferred_element_type=jnp.float32)
    cnt_part = lax.dot_general(P, jnp.ones((R, 8), jnp.float32),
                               (((0,), (0,)), ((), ())),
                               preferred_element_type=jnp.float32)

    @pl.when(i == 0)
    def _():
        s_acc[...] = jnp.zeros_like(s_acc)
        cnt_acc[...] = jnp.zeros_like(cnt_acc)

    s_acc[...] += s_part
    cnt_acc[...] += cnt_part

    @pl.when(i == GRID - 1)
    def _():
        cnt = cnt_acc[...][:, :1]
        pooled = s_acc[...] / jnp.maximum(cnt, 1.0)
        logits = jnp.dot(pooled, wl_ref[...],
                         preferred_element_type=jnp.float32) + bl_ref[...]
        m = jnp.max(logits, axis=1, keepdims=True)
        e = jnp.exp(logits - m)
        out_ref[...] = e / jnp.sum(e, axis=1, keepdims=True)


def _fin(deg_parts, agg, y2, b2, batch64, Wl, bl):
    return pl.pallas_call(
        _fin_body,
        grid=(GRID,),
        in_specs=[
            pl.BlockSpec((NC, 1, R, DW), lambda i: (0, i, 0, 0)),
            pl.BlockSpec((NC, R, H), lambda i: (0, i, 0)),
            pl.BlockSpec((R, H), lambda i: (i, 0)),
            pl.BlockSpec((1, H), lambda i: (0, 0)),
            pl.BlockSpec((R, G), lambda i: (i, 0)),
            pl.BlockSpec((H, C), lambda i: (0, 0)),
            pl.BlockSpec((1, C), lambda i: (0, 0)),
        ],
        out_specs=pl.BlockSpec((G, C), lambda i: (0, 0)),
        out_shape=jax.ShapeDtypeStruct((G, C), jnp.float32),
        scratch_shapes=[
            pltpu.VMEM((G, H), jnp.float32),
            pltpu.VMEM((G, 8), jnp.float32),
        ],
    )(deg_parts, agg, y2, b2.reshape(1, H), batch64, Wl, bl.reshape(1, C))


def kernel(x, edge_index, edge_attr, batch, W1, b1, W2, b2, Wl, bl):
    del edge_attr
    xpad = jnp.zeros((N_PAD, F_IN), jnp.float32).at[:N].set(x)
    fill = jnp.full((E_PAD - E,), N, jnp.int32)
    src = jnp.concatenate([edge_index[0], fill]).reshape(NW, NCHUNK, EC)
    dst = jnp.concatenate([edge_index[1], fill]).reshape(NW, NCHUNK, EC)
    batch_pad = jnp.concatenate([batch, jnp.full((N_PAD - N,), G, jnp.int32)])
    batch64 = jnp.broadcast_to(batch_pad[:, None], (N_PAD, G))

    xw = _mmxw(xpad, W1)
    deg_parts = _deg(dst).reshape(NC, GRID, R, DW)
    y1 = _scale(xw, deg_parts)
    agg1 = _agg(y1, src, dst)
    y2 = _mid(deg_parts, agg1, y1, b1, W2)
    agg2 = _agg(y2, src, dst)
    return _fin(deg_parts, agg2, y2, b2, batch64, Wl, bl)


# speedup vs baseline: 1.0090x; 1.0090x over previous
import functools

import jax
import jax.numpy as jnp
from jax import lax
from jax.experimental import pallas as pl
from jax.experimental.pallas import tpu as pltpu
from jax.experimental.pallas import tpu_sc as plsc

N = 10000
E = 320000
F_IN = 128
H = 64
C = 40
G = 64

NC = 2
NS = 16
NW = NC * NS

N_PAD = 10240
NR = N_PAD // NS
EC = 128
NCHUNK = 79
EWP = NCHUNK * EC
E_PAD = NW * EWP

R = 512
GRID = N_PAD // R

_mesh = plsc.VectorSubcoreMesh(core_axis_name="c", subcore_axis_name="s")
_sc_params = pltpu.CompilerParams(use_tc_tiling_on_sc=False)


DW = 16


@functools.partial(
    pl.kernel,
    mesh=_mesh,
    out_type=jax.ShapeDtypeStruct((NC, N_PAD, DW), jnp.float32),
    scratch_types=[
        pltpu.VMEM((NCHUNK, EC), jnp.int32),
        pltpu.VMEM((EC, DW), jnp.float32),
        pltpu.VMEM((NR, DW), jnp.float32),
        pltpu.VMEM_SHARED((N_PAD, DW), jnp.float32),
    ],
    compiler_params=_sc_params,
)
def _deg(dst_hbm, out_hbm, dstv, onesb, zbuf, acc):
    cid = lax.axis_index("c")
    sid = lax.axis_index("s")
    wid = sid * NC + cid
    pltpu.sync_copy(dst_hbm.at[wid], dstv)
    zv = jnp.zeros((16,), jnp.float32)
    ones = jnp.ones((16,), jnp.float32)

    def fill(r, carry):
        onesb[r, pl.ds(0, 16)] = ones
        return carry

    lax.fori_loop(0, EC, fill, 0)

    def zfill(r, carry):
        zbuf[r, pl.ds(0, 16)] = zv
        return carry

    lax.fori_loop(0, NR, zfill, 0)
    base = pl.multiple_of(sid * NR, 128)
    pltpu.sync_copy(zbuf, acc.at[pl.ds(base, NR)])
    plsc.subcore_barrier()

    def chunk(j, carry):
        pltpu.sync_copy(onesb, acc.at[dstv.at[j]], add=True)
        return carry

    lax.fori_loop(0, NCHUNK, chunk, 0)
    plsc.subcore_barrier()
    pltpu.sync_copy(acc.at[pl.ds(base, NR)], zbuf)
    pltpu.sync_copy(zbuf, out_hbm.at[cid, pl.ds(base, NR)])


@functools.partial(
    pl.kernel,
    mesh=_mesh,
    out_type=jax.ShapeDtypeStruct((NC, N_PAD, H), jnp.bfloat16),
    scratch_types=[
        pltpu.VMEM((NCHUNK, EC), jnp.int32),
        pltpu.VMEM((NCHUNK, EC), jnp.int32),
        pltpu.VMEM((2, EC, H), jnp.bfloat16),
        pltpu.VMEM((EC, H), jnp.bfloat16),
        pltpu.VMEM_SHARED((N_PAD, H), jnp.bfloat16),
        pltpu.VMEM_SHARED((N_PAD, H), jnp.bfloat16),
        pltpu.SemaphoreType.DMA,
    ],
    compiler_params=_sc_params,
)
def _agg(y_hbm, src_hbm, dst_hbm, out_hbm, srcv, dstv, rows, zbuf, acc, ysh,
         gsem):
    cid = lax.axis_index("c")
    sid = lax.axis_index("s")
    wid = sid * NC + cid
    pltpu.sync_copy(src_hbm.at[wid], srcv)
    pltpu.sync_copy(dst_hbm.at[wid], dstv)

    zv = jnp.zeros((32,), jnp.bfloat16)

    def zrow(r, carry):
        zbuf[r, pl.ds(0, 32)] = zv
        zbuf[r, pl.ds(32, 32)] = zv
        return carry

    lax.fori_loop(0, EC, zrow, 0)
    base = pl.multiple_of(sid * NR, 128)
    for k in range(NR // EC):
        pltpu.sync_copy(zbuf, acc.at[pl.ds(base + k * EC, EC)])
    pltpu.sync_copy(y_hbm.at[pl.ds(base, NR)], ysh.at[pl.ds(base, NR)])
    plsc.subcore_barrier()

    def chunk(j, carry):
        pltpu.async_copy(ysh.at[srcv.at[j]], rows.at[0], gsem).wait()
        pltpu.sync_copy(rows.at[0], acc.at[dstv.at[j]], add=True)
        return carry

    lax.fori_loop(0, NCHUNK, chunk, 0)
    plsc.subcore_barrier()

    pltpu.sync_copy(acc.at[pl.ds(base, NR)], out_hbm.at[cid, pl.ds(base, NR)])


def _dinv_block(deg_ref):
    a = deg_ref[...]
    degs = a[0, 0, :, :1] + a[1, 0, :, :1]
    return lax.rsqrt(1.0 + degs)


def _mm1_body(deg_ref, x_ref, w_ref, y_ref):
    dinv = _dinv_block(deg_ref)
    xw = jnp.dot(x_ref[...], w_ref[...], preferred_element_type=jnp.float32)
    y_ref[...] = (xw * dinv).astype(jnp.bfloat16)


def _mm1(xpad, W1, deg_parts):
    return pl.pallas_call(
        _mm1_body,
        grid=(GRID,),
        in_specs=[
            pl.BlockSpec((NC, 1, R, DW), lambda i: (0, i, 0, 0)),
            pl.BlockSpec((R, F_IN), lambda i: (i, 0)),
            pl.BlockSpec((F_IN, H), lambda i: (0, 0)),
        ],
        out_specs=pl.BlockSpec((R, H), lambda i: (i, 0)),
        out_shape=jax.ShapeDtypeStruct((N_PAD, H), jnp.bfloat16),
    )(deg_parts, xpad, W1)


def _mid_body(deg_ref, agg_ref, y_ref, b_ref, w_ref, y2_ref):
    dinv = _dinv_block(deg_ref)
    a = agg_ref[...].astype(jnp.float32)
    y = y_ref[...].astype(jnp.float32)
    h = (a[0] + a[1] + y) * dinv + b_ref[...]
    h = jnp.maximum(h, 0.0)
    y2_ref[...] = (jnp.dot(h, w_ref[...],
                           preferred_element_type=jnp.float32)
                   * dinv).astype(jnp.bfloat16)


def _mid(deg_parts, agg, y1, b1, W2):
    return pl.pallas_call(
        _mid_body,
        grid=(GRID,),
        in_specs=[
            pl.BlockSpec((NC, 1, R, DW), lambda i: (0, i, 0, 0)),
            pl.BlockSpec((NC, R, H), lambda i: (0, i, 0)),
            pl.BlockSpec((R, H), lambda i: (i, 0)),
            pl.BlockSpec((1, H), lambda i: (0, 0)),
            pl.BlockSpec((H, H), lambda i: (0, 0)),
        ],
        out_specs=pl.BlockSpec((R, H), lambda i: (i, 0)),
        out_shape=jax.ShapeDtypeStruct((N_PAD, H), jnp.bfloat16),
    )(deg_parts, agg, y1, b1.reshape(1, H), W2)


def _fin_body(deg_ref, agg_ref, y_ref, b_ref, batch_ref, wl_ref, bl_ref,
              out_ref, s_acc, cnt_acc):
    i = pl.program_id(0)
    dinv = _dinv_block(deg_ref)
    a = agg_ref[...].astype(jnp.float32)
    y = y_ref[...].astype(jnp.float32)
    h = (a[0] + a[1] + y) * dinv + b_ref[...]
    h = jnp.maximum(h, 0.0)
    seg = lax.broadcasted_iota(jnp.int32, (1, G), 1)
    P = (batch_ref[...] == seg).astype(jnp.float32)
    s_part = lax.dot_general(P, h, (((0,), (0,)), ((), ())),
                             preferred_element_type=jnp.float32)
    cnt_part = lax.dot_general(P, jnp.ones((R, 8), jnp.float32),
                               (((0,), (0,)), ((), ())),
                               preferred_element_type=jnp.float32)

    @pl.when(i == 0)
    def _():
        s_acc[...] = jnp.zeros_like(s_acc)
        cnt_acc[...] = jnp.zeros_like(cnt_acc)

    s_acc[...] += s_part
    cnt_acc[...] += cnt_part

    @pl.when(i == GRID - 1)
    def _():
        cnt = cnt_acc[...][:, :1]
        pooled = s_acc[...] / jnp.maximum(cnt, 1.0)
        logits = jnp.dot(pooled, wl_ref[...],
                         preferred_element_type=jnp.float32) + bl_ref[...]
        m = jnp.max(logits, axis=1, keepdims=True)
        e = jnp.exp(logits - m)
        out_ref[...] = e / jnp.sum(e, axis=1, keepdims=True)


def _fin(deg_parts, agg, y2, b2, batch64, Wl, bl):
    return pl.pallas_call(
        _fin_body,
        grid=(GRID,),
        in_specs=[
            pl.BlockSpec((NC, 1, R, DW), lambda i: (0, i, 0, 0)),
            pl.BlockSpec((NC, R, H), lambda i: (0, i, 0)),
            pl.BlockSpec((R, H), lambda i: (i, 0)),
            pl.BlockSpec((1, H), lambda i: (0, 0)),
            pl.BlockSpec((R, G), lambda i: (i, 0)),
            pl.BlockSpec((H, C), lambda i: (0, 0)),
            pl.BlockSpec((1, C), lambda i: (0, 0)),
        ],
        out_specs=pl.BlockSpec((G, C), lambda i: (0, 0)),
        out_shape=jax.ShapeDtypeStruct((G, C), jnp.float32),
        scratch_shapes=[
            pltpu.VMEM((G, H), jnp.float32),
            pltpu.VMEM((G, 8), jnp.float32),
        ],
    )(deg_parts, agg, y2, b2.reshape(1, H), batch64, Wl, bl.reshape(1, C))


def kernel(x, edge_index, edge_attr, batch, W1, b1, W2, b2, Wl, bl):
    del edge_attr
    xpad = jnp.zeros((N_PAD, F_IN), jnp.float32).at[:N].set(x)
    fill = jnp.full((E_PAD - E,), N, jnp.int32)
    src = jnp.concatenate([edge_index[0], fill]).reshape(NW, NCHUNK, EC)
    dst = jnp.concatenate([edge_index[1], fill]).reshape(NW, NCHUNK, EC)
    batch_pad = jnp.concatenate([batch, jnp.full((N_PAD - N,), G, jnp.int32)])
    batch64 = jnp.broadcast_to(batch_pad[:, None], (N_PAD, G))

    deg_parts = _deg(dst).reshape(NC, GRID, R, DW)
    y1 = _mm1(xpad, W1, deg_parts)
    agg1 = _agg(y1, src, dst)
    y2 = _mid(deg_parts, agg1, y1, b1, W2)
    agg2 = _agg(y2, src, dst)
    return _fin(deg_parts, agg2, y2, b2, batch64, Wl, bl)
